# Initial kernel scaffold; baseline (speedup 1.0000x reference)
import jax
import jax.numpy as jnp
from jax.experimental import pallas as pl


def kernel(x, x1, emb, W, b):
    raise NotImplementedError("write your pallas kernel here")


# speedup vs baseline: 4.6699x; 4.6699x over previous
import functools

import jax
import jax.numpy as jnp
from jax import lax
from jax.experimental import pallas as pl
from jax.experimental.pallas import tpu as pltpu
from jax.experimental.pallas import tpu_sc as plsc

B = 16384
V = 49
VP = 64
DE = 3
C = 4
DO = 3
L = 16
NW = 32
BW = B // NW

_mesh = plsc.VectorSubcoreMesh(core_axis_name="c", subcore_axis_name="s")


@functools.partial(
    pl.kernel,
    out_type=(
        jax.ShapeDtypeStruct((B, DO), jnp.float32),
        jax.ShapeDtypeStruct((B, DE), jnp.float32),
    ),
    mesh=_mesh,
    compiler_params=pltpu.CompilerParams(
        needs_layout_passes=False, use_tc_tiling_on_sc=False
    ),
    scratch_types=[
        pltpu.VMEM((BW, C), jnp.int32),
        pltpu.VMEM((BW,), jnp.int32),
        pltpu.VMEM((VP, DE), jnp.float32),
        pltpu.VMEM((48,), jnp.float32),
        pltpu.VMEM((L,), jnp.float32),
        pltpu.VMEM((C, VP, DO), jnp.float32),
        pltpu.VMEM((BW, DO), jnp.float32),
        pltpu.VMEM((BW, DE), jnp.float32),
    ],
)
def _cbow_sc(x_hbm, x1_hbm, emb_hbm, w_hbm, b_hbm, y_hbm, y1_hbm,
             x_v, x1_v, emb_v, w_v, b_v, t_v, y_v, y1_v):
    nc = _mesh.num_cores
    wid = lax.axis_index("s") * nc + lax.axis_index("c")
    base = wid * BW

    pltpu.sync_copy(x_hbm.at[pl.ds(base, BW), :], x_v)
    pltpu.sync_copy(x1_hbm.at[pl.ds(base, BW)], x1_v)
    pltpu.sync_copy(emb_hbm, emb_v.at[pl.ds(0, V), :])
    pltpu.sync_copy(w_hbm, w_v)
    pltpu.sync_copy(b_hbm, b_v)

    iota = lax.iota(jnp.int32, L)
    cconst = [jnp.full((L,), c, jnp.int32) for c in range(C)]
    jconst = [jnp.full((L,), j, jnp.int32) for j in range(max(DO, DE))]

    w_vec = [w_v[pl.ds(k * L, L)] for k in range(3)]
    b_vec = b_v[...]

    def w_scalar(j, k):
        idx = 12 * j + k
        return w_vec[idx // L][idx % L]

    for vg in range(VP // L):
        vv = vg * L + iota
        m = vv < V
        e = [plsc.load_gather(emb_v, [vv, jconst[d]]) for d in range(DE)]
        for c in range(C):
            for j in range(DO):
                acc = e[0] * w_scalar(j, 3 * c + 0)
                acc = acc + e[1] * w_scalar(j, 3 * c + 1)
                acc = acc + e[2] * w_scalar(j, 3 * c + 2)
                if c == 0:
                    acc = acc + b_vec[j]
                plsc.store_scatter(t_v, [cconst[c], vv, jconst[j]], acc, mask=m)

    def group(g, carry):
        rows = g * L + iota
        xc = [plsc.load_gather(x_v, [rows, cconst[c]]) for c in range(C)]
        for j in range(DO):
            acc = plsc.load_gather(t_v, [cconst[0], xc[0], jconst[j]])
            for c in range(1, C):
                acc = acc + plsc.load_gather(t_v, [cconst[c], xc[c], jconst[j]])
            plsc.store_scatter(y_v, [rows, jconst[j]], acc)
        x1c = x1_v[pl.ds(g * L, L)]
        for j in range(DE):
            plsc.store_scatter(
                y1_v, [rows, jconst[j]], plsc.load_gather(emb_v, [x1c, jconst[j]])
            )
        return carry

    lax.fori_loop(0, BW // L, group, 0)

    pltpu.sync_copy(y_v, y_hbm.at[pl.ds(base, BW), :])
    pltpu.sync_copy(y1_v, y1_hbm.at[pl.ds(base, BW), :])


def kernel(x, x1, emb, W, b):
    w_flat = jnp.pad(W.reshape(-1), (0, 48 - C * DO * DE))
    b_pad = jnp.pad(b, (0, L - DO))
    return _cbow_sc(x.astype(jnp.int32), x1.astype(jnp.int32), emb, w_flat, b_pad)
